# baseline (speedup 1.0000x reference)
import functools

import jax
import jax.numpy as jnp
from jax import lax
from jax.experimental import pallas as pl
from jax.experimental.pallas import tpu as pltpu
from jax.experimental.pallas import tpu_sc as plsc

B = 4
L = 8192
D = 64
N_IDX = B * L

_info = plsc.get_sparse_core_info()
NC, NS = _info.num_cores, _info.num_subcores
NW = NC * NS
B_W = N_IDX // NW
CH = 128
NCH = B_W // CH

_mesh = plsc.VectorSubcoreMesh(core_axis_name="c", subcore_axis_name="s")


@functools.partial(
    pl.kernel,
    mesh=_mesh,
    compiler_params=pltpu.CompilerParams(use_tc_tiling_on_sc=True),
    out_type=jax.ShapeDtypeStruct((N_IDX, D), jnp.float32),
    scratch_types=[
        pltpu.VMEM((B_W,), jnp.int32),
        pltpu.VMEM((B_W,), jnp.int32),
        pltpu.VMEM((2, CH, 2 * D), jnp.float32),
        pltpu.VMEM((2, CH, D), jnp.float32),
        pltpu.SemaphoreType.DMA,
        pltpu.SemaphoreType.DMA,
        pltpu.SemaphoreType.DMA,
        pltpu.SemaphoreType.DMA,
    ],
)
def _embed_gather(idx_hbm, table2_hbm, out_hbm, idx_v, idx2_v, buf_v, sel_v,
                  g_sem0, g_sem1, o_sem0, o_sem1):
    wid = lax.axis_index("s") * NC + lax.axis_index("c")
    base = wid * B_W
    b = wid // (L // B_W)
    l0 = (wid % (L // B_W)) * B_W
    pltpu.sync_copy(idx_hbm.at[b, pl.ds(l0, B_W)], idx_v)

    def shift_pair(k, _):
        idx2_v[pl.ds(k * 16, 16)] = (
            lax.shift_right_logical(idx_v[pl.ds(k * 16, 16)], 1)
        )
        return _

    lax.fori_loop(0, B_W // 16, shift_pair, 0)

    g_sems = (g_sem0, g_sem1)
    o_sems = (o_sem0, o_sem1)

    def gather(j):
        pltpu.make_async_copy(
            table2_hbm.at[idx2_v.at[pl.ds(j * CH, CH)]],
            buf_v.at[j % 2],
            g_sems[j % 2],
        ).start()

    def gather_wait(j):
        pltpu.make_async_copy(
            table2_hbm.at[idx2_v.at[pl.ds(0, CH)]],
            buf_v.at[j % 2],
            g_sems[j % 2],
        ).wait()

    def select(j):
        jb = j % 2

        def sel_group(g, _):
            p16 = idx_v[pl.ds(j * CH + g * 16, 16)] & 1
            for l in range(16):
                i = g * 16 + l
                off = p16[l] * D
                for c0 in range(0, D, 16):
                    sel_v[jb, i, pl.ds(c0, 16)] = (
                        buf_v[jb, i, pl.ds(off + c0, 16)]
                    )
            return _

        lax.fori_loop(0, CH // 16, sel_group, 0)

    def out_start(j):
        pltpu.make_async_copy(
            sel_v.at[j % 2],
            out_hbm.at[pl.ds(base + j * CH, CH)],
            o_sems[j % 2],
        ).start()

    def out_wait(j):
        pltpu.make_async_copy(
            sel_v.at[j % 2],
            out_hbm.at[pl.ds(base, CH)],
            o_sems[j % 2],
        ).wait()

    gather(0)
    for j in range(NCH):
        if j + 1 < NCH:
            gather(j + 1)
        gather_wait(j)
        if j >= 2:
            out_wait(j - 2)
        select(j)
        out_start(j)

    out_wait(NCH - 2)
    out_wait(NCH - 1)


def kernel(x_BL, table):
    table2 = table.reshape(500000, 2 * D)
    out = _embed_gather(x_BL.astype(jnp.int32), table2)
    return out.reshape(B, L, D)
